# baseline (speedup 1.0000x reference)
import functools

import jax
import jax.numpy as jnp
from jax.experimental import pallas as pl
from jax.experimental.pallas import tpu as pltpu

NUM_EXPERTS = 8
TOP_K = 2
HIDDEN_DIM = 1024
FFN_DIM = 4096
BETA = 0.5

TOKENS = 512
F_TILE = 1024
F_TILES = FFN_DIM // F_TILE
BT = 128


def _router_kernel(x_ref, gate_ref, logits_ref, wt_ref, rank_ref, cnt_ref):
    x = x_ref[...]
    logits = jax.lax.dot_general(
        x, gate_ref[...], (((1,), (1,)), ((), ())),
        preferred_element_type=jnp.float32)
    logits_ref[...] = logits
    probs = jax.nn.softmax(logits, axis=-1)
    eidx = jax.lax.broadcasted_iota(jnp.int32, probs.shape, 1)
    m1 = jnp.max(probs, axis=-1, keepdims=True)
    i1 = jnp.argmax(probs, axis=-1, keepdims=True)
    masked = jnp.where(eidx == i1, -jnp.inf, probs)
    m2 = jnp.max(masked, axis=-1, keepdims=True)
    i2 = jnp.argmax(masked, axis=-1, keepdims=True)
    w2v = jnp.where(m2 < BETA * m1, 0.0, m2)
    s = m1 + w2v
    wtok = jnp.where(eidx == i1, m1 / s,
                     jnp.where(eidx == i2, w2v / s, 0.0))

    r_iota = jax.lax.broadcasted_iota(jnp.int32, (TOKENS, TOKENS), 0)
    c_iota = jax.lax.broadcasted_iota(jnp.int32, (TOKENS, TOKENS), 1)
    ident = (r_iota == c_iota).astype(jnp.float32)
    wt = jax.lax.dot_general(wtok, ident, (((0,), (0,)), ((), ())),
                             preferred_element_type=jnp.float32,
                             precision=jax.lax.Precision.HIGHEST)
    wt_ref[...] = wt
    a_t = (wt > 0.0).astype(jnp.float32)
    upper = (r_iota < c_iota).astype(jnp.float32)
    rank = jax.lax.dot_general(a_t, upper, (((1,), (0,)), ((), ())),
                               preferred_element_type=jnp.float32,
                               precision=jax.lax.Precision.HIGHEST)
    rank_ref[...] = rank
    ones_row = jnp.ones((1, TOKENS), jnp.float32)
    cnt_ref[...] = jax.lax.dot_general(
        ones_row, a_t, (((1,), (1,)), ((), ())),
        preferred_element_type=jnp.float32,
        precision=jax.lax.Precision.HIGHEST).astype(jnp.int32)


def _ffn_kernel(cnt_ref, x_ref, wt_ref, rank_ref, w1_ref, w2_ref, w3_ref,
                out_ref, xg_ref, og_ref):
    e = pl.program_id(0)
    f = pl.program_id(1)

    @pl.when((e == 0) & (f == 0))
    def _init():
        out_ref[...] = jnp.zeros_like(out_ref)

    cnt_e = cnt_ref[e]
    nb = (cnt_e + BT - 1) // BT

    sub = jax.lax.broadcasted_iota(jnp.int32, (NUM_EXPERTS, TOKENS), 0)
    w_e = jnp.sum(jnp.where(sub == e, wt_ref[...], 0.0), axis=0,
                  keepdims=True)
    r_e = jnp.sum(jnp.where(sub == e, rank_ref[...], 0.0), axis=0,
                  keepdims=True)
    assigned = w_e > 0.0

    blk_iota = jax.lax.broadcasted_iota(jnp.int32, (BT, TOKENS), 0)
    r_e_i = r_e.astype(jnp.int32)

    def onehot(b):
        return ((r_e_i == blk_iota + b * BT) & assigned).astype(jnp.float32)

    @pl.when(f == 0)
    def _gather():
        x = x_ref[...]

        def body(b, carry):
            m = onehot(b)
            xg_ref[pl.ds(b * BT, BT), :] = jax.lax.dot_general(
                m, x, (((1,), (0,)), ((), ())),
                preferred_element_type=jnp.float32)
            return carry

        jax.lax.fori_loop(0, nb, body, 0)

    w1t = w1_ref[0]
    w3t = w3_ref[0]
    w2t = w2_ref[0]
    dn = (((1,), (1,)), ((), ()))

    def ffn_body(b, carry):
        x_gb = xg_ref[pl.ds(b * BT, BT), :]
        a = jax.lax.dot_general(x_gb, w1t, dn, preferred_element_type=jnp.float32)
        c = jax.lax.dot_general(x_gb, w3t, dn, preferred_element_type=jnp.float32)
        h = jax.nn.silu(a) * c
        part = jax.lax.dot_general(h, w2t, dn,
                                   preferred_element_type=jnp.float32)

        @pl.when(f == 0)
        def _set():
            og_ref[pl.ds(b * BT, BT), :] = part

        @pl.when(f != 0)
        def _acc():
            og_ref[pl.ds(b * BT, BT), :] += part

        return carry

    jax.lax.fori_loop(0, nb, ffn_body, 0)

    @pl.when(f == F_TILES - 1)
    def _scatter():
        def body(b, carry):
            m = onehot(b)
            w_row = jnp.sum(m * w_e, axis=1, keepdims=True)
            og = og_ref[pl.ds(b * BT, BT), :]
            out_ref[...] += jax.lax.dot_general(
                m, og * w_row, (((0,), (0,)), ((), ())),
                preferred_element_type=jnp.float32)
            return carry

        jax.lax.fori_loop(0, nb, body, 0)


@jax.jit
def kernel(hidden_states, gate_w, w1, w2, w3):
    batch, seq, hid = hidden_states.shape
    x = hidden_states.reshape(TOKENS, hid)

    logits, wt, rank, cnt = pl.pallas_call(
        _router_kernel,
        in_specs=[
            pl.BlockSpec((TOKENS, hid), lambda: (0, 0)),
            pl.BlockSpec((NUM_EXPERTS, hid), lambda: (0, 0)),
        ],
        out_specs=[
            pl.BlockSpec((TOKENS, NUM_EXPERTS), lambda: (0, 0)),
            pl.BlockSpec((NUM_EXPERTS, TOKENS), lambda: (0, 0)),
            pl.BlockSpec((NUM_EXPERTS, TOKENS), lambda: (0, 0)),
            pl.BlockSpec((1, NUM_EXPERTS), lambda: (0, 0)),
        ],
        out_shape=[
            jax.ShapeDtypeStruct((TOKENS, NUM_EXPERTS), jnp.float32),
            jax.ShapeDtypeStruct((NUM_EXPERTS, TOKENS), jnp.float32),
            jax.ShapeDtypeStruct((NUM_EXPERTS, TOKENS), jnp.float32),
            jax.ShapeDtypeStruct((1, NUM_EXPERTS), jnp.int32),
        ],
    )(x, gate_w)

    grid_spec = pltpu.PrefetchScalarGridSpec(
        num_scalar_prefetch=1,
        grid=(NUM_EXPERTS, F_TILES),
        in_specs=[
            pl.BlockSpec((TOKENS, hid), lambda e, f, c: (0, 0)),
            pl.BlockSpec((NUM_EXPERTS, TOKENS), lambda e, f, c: (0, 0)),
            pl.BlockSpec((NUM_EXPERTS, TOKENS), lambda e, f, c: (0, 0)),
            pl.BlockSpec((1, F_TILE, hid), lambda e, f, c: (e, f, 0)),
            pl.BlockSpec((1, hid, F_TILE), lambda e, f, c: (e, 0, f)),
            pl.BlockSpec((1, F_TILE, hid), lambda e, f, c: (e, f, 0)),
        ],
        out_specs=pl.BlockSpec((TOKENS, hid), lambda e, f, c: (0, 0)),
        scratch_shapes=[
            pltpu.VMEM((TOKENS, HIDDEN_DIM), jnp.float32),
            pltpu.VMEM((TOKENS, HIDDEN_DIM), jnp.float32),
        ],
    )
    out = pl.pallas_call(
        _ffn_kernel,
        grid_spec=grid_spec,
        out_shape=jax.ShapeDtypeStruct((TOKENS, hid), jnp.float32),
    )(cnt.reshape(NUM_EXPERTS), x, wt, rank, w1, w2, w3)

    return out.reshape(batch, seq, hid), logits


# speedup vs baseline: 1.0152x; 1.0152x over previous
import functools

import jax
import jax.numpy as jnp
from jax.experimental import pallas as pl
from jax.experimental.pallas import tpu as pltpu

NUM_EXPERTS = 8
TOP_K = 2
HIDDEN_DIM = 1024
FFN_DIM = 4096
BETA = 0.5

TOKENS = 512
F_TILE = 1024
F_TILES = FFN_DIM // F_TILE
BT = 128


def _moe_kernel(x_ref, gate_ref, w1_ref, w2_ref, w3_ref,
                out_ref, logits_ref,
                wt_ref, rank_ref, cnt_ref, xg_ref, og_ref):
    e = pl.program_id(0)
    f = pl.program_id(1)

    @pl.when((e == 0) & (f == 0))
    def _router():
        x = x_ref[...]
        logits = jax.lax.dot_general(
            x, gate_ref[...], (((1,), (1,)), ((), ())),
            preferred_element_type=jnp.float32)
        logits_ref[...] = logits
        probs = jax.nn.softmax(logits, axis=-1)
        eidx = jax.lax.broadcasted_iota(jnp.int32, probs.shape, 1)
        m1 = jnp.max(probs, axis=-1, keepdims=True)
        i1 = jnp.argmax(probs, axis=-1, keepdims=True)
        masked = jnp.where(eidx == i1, -jnp.inf, probs)
        m2 = jnp.max(masked, axis=-1, keepdims=True)
        i2 = jnp.argmax(masked, axis=-1, keepdims=True)
        w2v = jnp.where(m2 < BETA * m1, 0.0, m2)
        s = m1 + w2v
        wtok = jnp.where(eidx == i1, m1 / s,
                         jnp.where(eidx == i2, w2v / s, 0.0))
        r_iota = jax.lax.broadcasted_iota(jnp.int32, (TOKENS, TOKENS), 0)
        c_iota = jax.lax.broadcasted_iota(jnp.int32, (TOKENS, TOKENS), 1)
        ident = (r_iota == c_iota).astype(jnp.float32)
        wt = jax.lax.dot_general(wtok, ident, (((0,), (0,)), ((), ())),
                                 preferred_element_type=jnp.float32,
                                 precision=jax.lax.Precision.HIGHEST)
        wt_ref[...] = wt
        a_t = (wt > 0.0).astype(jnp.float32)
        upper = (r_iota < c_iota).astype(jnp.float32)
        rank_ref[...] = jax.lax.dot_general(
            a_t, upper, (((1,), (0,)), ((), ())),
            preferred_element_type=jnp.float32,
            precision=jax.lax.Precision.HIGHEST)
        ones_cols = jnp.ones((TOKENS, 128), jnp.float32)
        cnt_ref[...] = jax.lax.dot_general(
            a_t, ones_cols, (((1,), (0,)), ((), ())),
            preferred_element_type=jnp.float32,
            precision=jax.lax.Precision.HIGHEST).astype(jnp.int32)
        out_ref[...] = jnp.zeros_like(out_ref)

    cnt_e = cnt_ref[e, 0]
    nb = (cnt_e + BT - 1) // BT

    sub = jax.lax.broadcasted_iota(jnp.int32, (NUM_EXPERTS, TOKENS), 0)
    w_e = jnp.sum(jnp.where(sub == e, wt_ref[...], 0.0), axis=0,
                  keepdims=True)
    r_e = jnp.sum(jnp.where(sub == e, rank_ref[...], 0.0), axis=0,
                  keepdims=True)
    assigned = w_e > 0.0

    blk_iota = jax.lax.broadcasted_iota(jnp.int32, (BT, TOKENS), 0)
    r_e_i = r_e.astype(jnp.int32)

    def onehot(b):
        return ((r_e_i == blk_iota + b * BT) & assigned).astype(jnp.float32)

    @pl.when(f == 0)
    def _gather():
        x = x_ref[...]

        def body(b, carry):
            m = onehot(b)
            xg_ref[pl.ds(b * BT, BT), :] = jax.lax.dot_general(
                m, x, (((1,), (0,)), ((), ())),
                preferred_element_type=jnp.float32)
            return carry

        jax.lax.fori_loop(0, nb, body, 0)

    w1t = w1_ref[0].astype(jnp.bfloat16)
    w3t = w3_ref[0].astype(jnp.bfloat16)
    w2t = w2_ref[0].astype(jnp.bfloat16)
    dn = (((1,), (1,)), ((), ()))

    def ffn_body(b, carry):
        x_gb = xg_ref[pl.ds(b * BT, BT), :].astype(jnp.bfloat16)
        a = jax.lax.dot_general(x_gb, w1t, dn, preferred_element_type=jnp.float32)
        c = jax.lax.dot_general(x_gb, w3t, dn, preferred_element_type=jnp.float32)
        h = jax.nn.silu(a) * c
        part = jax.lax.dot_general(h.astype(jnp.bfloat16), w2t, dn,
                                   preferred_element_type=jnp.float32)

        @pl.when(f == 0)
        def _set():
            og_ref[pl.ds(b * BT, BT), :] = part

        @pl.when(f != 0)
        def _acc():
            og_ref[pl.ds(b * BT, BT), :] += part

        return carry

    jax.lax.fori_loop(0, nb, ffn_body, 0)

    @pl.when(f == F_TILES - 1)
    def _scatter():
        def body(b, carry):
            m = onehot(b)
            w_row = jnp.sum(m * w_e, axis=1, keepdims=True)
            og = og_ref[pl.ds(b * BT, BT), :]
            out_ref[...] += jax.lax.dot_general(
                m, og * w_row, (((0,), (0,)), ((), ())),
                preferred_element_type=jnp.float32)
            return carry

        jax.lax.fori_loop(0, nb, body, 0)


@jax.jit
def kernel(hidden_states, gate_w, w1, w2, w3):
    batch, seq, hid = hidden_states.shape
    x = hidden_states.reshape(TOKENS, hid)

    out, logits = pl.pallas_call(
        _moe_kernel,
        grid=(NUM_EXPERTS, F_TILES),
        in_specs=[
            pl.BlockSpec((TOKENS, hid), lambda e, f: (0, 0)),
            pl.BlockSpec((NUM_EXPERTS, hid), lambda e, f: (0, 0)),
            pl.BlockSpec((1, F_TILE, hid), lambda e, f: (e, f, 0)),
            pl.BlockSpec((1, hid, F_TILE), lambda e, f: (e, 0, f)),
            pl.BlockSpec((1, F_TILE, hid), lambda e, f: (e, f, 0)),
        ],
        out_specs=[
            pl.BlockSpec((TOKENS, hid), lambda e, f: (0, 0)),
            pl.BlockSpec((TOKENS, NUM_EXPERTS), lambda e, f: (0, 0)),
        ],
        out_shape=[
            jax.ShapeDtypeStruct((TOKENS, hid), jnp.float32),
            jax.ShapeDtypeStruct((TOKENS, NUM_EXPERTS), jnp.float32),
        ],
        scratch_shapes=[
            pltpu.VMEM((NUM_EXPERTS, TOKENS), jnp.float32),
            pltpu.VMEM((NUM_EXPERTS, TOKENS), jnp.float32),
            pltpu.VMEM((NUM_EXPERTS, 128), jnp.int32),
            pltpu.VMEM((TOKENS, HIDDEN_DIM), jnp.float32),
            pltpu.VMEM((TOKENS, HIDDEN_DIM), jnp.float32),
        ],
    )(x, gate_w, w1, w2, w3)

    return out.reshape(batch, seq, hid), logits
